# baseline (speedup 1.0000x reference)
import jax
import jax.numpy as jnp
from jax import lax
from jax.experimental import pallas as pl
from jax.experimental.pallas import tpu as pltpu
from jax.experimental.pallas import tpu_sc as plsc

NUM_CORES = 2
NUM_SUBCORES = 16
NUM_WORKERS = NUM_CORES * NUM_SUBCORES
CHUNK_B = 32


def _gather_kernel(table_hbm, ids_hbm, out_hbm, idx_v, rows_v, sem):
    n_batch, seq, _ = out_hbm.shape
    b_per_w = n_batch // NUM_WORKERS
    wid = lax.axis_index("s") * NUM_CORES + lax.axis_index("c")
    b0w = wid * b_per_w

    @pl.loop(0, b_per_w, step=CHUNK_B)
    def _(bo):
        b0 = b0w + bo
        pltpu.sync_copy(ids_hbm.at[pl.ds(b0 * seq, CHUNK_B * seq)], idx_v)
        pltpu.async_copy(table_hbm.at[idx_v], rows_v, sem).wait()
        handles = [
            pltpu.async_copy(
                rows_v.at[pl.ds(i * seq, seq)], out_hbm.at[b0 + i], sem
            )
            for i in range(CHUNK_B)
        ]
        for h in handles:
            h.wait()


def kernel(token_ids, embedding_table):
    batch, seq = token_ids.shape
    dim = embedding_table.shape[1]
    flat_ids = token_ids.reshape(-1).astype(jnp.int32)
    table16 = embedding_table.astype(jnp.bfloat16)

    mesh = plsc.VectorSubcoreMesh(core_axis_name="c", subcore_axis_name="s")
    k = pl.kernel(
        _gather_kernel,
        mesh=mesh,
        out_type=jax.ShapeDtypeStruct((batch, seq, dim), jnp.bfloat16),
        scratch_types=[
            pltpu.VMEM((CHUNK_B * seq,), jnp.int32),
            pltpu.VMEM((CHUNK_B * seq, dim), jnp.bfloat16),
            pltpu.SemaphoreType.DMA,
        ],
        compiler_params=pltpu.CompilerParams(use_tc_tiling_on_sc=False),
    )
    out16 = k(table16, flat_ids)
    return out16.astype(jnp.float32)


# speedup vs baseline: 1.3719x; 1.3719x over previous
import jax
import jax.numpy as jnp
from jax import lax
from jax.experimental import pallas as pl
from jax.experimental.pallas import tpu as pltpu
from jax.experimental.pallas import tpu_sc as plsc

NUM_CORES = 2
NUM_SUBCORES = 16
NUM_WORKERS = NUM_CORES * NUM_SUBCORES
CHUNK_B = 64


def _gather_kernel(table_hbm, ids_hbm, out_hbm, idx_v, rows_v, sem):
    n_batch, seq, _ = out_hbm.shape
    b_per_w = n_batch // NUM_WORKERS
    wid = lax.axis_index("s") * NUM_CORES + lax.axis_index("c")
    b0w = wid * b_per_w

    @pl.loop(0, b_per_w, step=CHUNK_B)
    def _(bo):
        b0 = b0w + bo
        pltpu.sync_copy(ids_hbm.at[pl.ds(b0 * seq, CHUNK_B * seq)], idx_v)
        pltpu.async_copy(table_hbm.at[idx_v], rows_v, sem).wait()
        handles = [
            pltpu.async_copy(
                rows_v.at[pl.ds(i * seq, seq)], out_hbm.at[b0 + i], sem
            )
            for i in range(CHUNK_B)
        ]
        for h in handles:
            h.wait()


def kernel(token_ids, embedding_table):
    batch, seq = token_ids.shape
    dim = embedding_table.shape[1]
    flat_ids = token_ids.reshape(-1).astype(jnp.int32)

    mesh = plsc.VectorSubcoreMesh(core_axis_name="c", subcore_axis_name="s")
    k = pl.kernel(
        _gather_kernel,
        mesh=mesh,
        out_type=jax.ShapeDtypeStruct((batch, seq, dim), embedding_table.dtype),
        scratch_types=[
            pltpu.VMEM((CHUNK_B * seq,), jnp.int32),
            pltpu.VMEM((CHUNK_B * seq, dim), jnp.float32),
            pltpu.SemaphoreType.DMA,
        ],
        compiler_params=pltpu.CompilerParams(use_tc_tiling_on_sc=False),
    )
    return k(embedding_table, flat_ids)
